# baseline (speedup 1.0000x reference)
import functools

import jax
import jax.numpy as jnp
from jax import lax
from jax.experimental import pallas as pl
from jax.experimental.pallas import tpu as pltpu
from jax.experimental.pallas import tpu_sc as plsc

_NUM_OBJS = 151
_L = 16


@functools.lru_cache(maxsize=None)
def _make_gather(B, V, D):
    info = plsc.get_sparse_core_info()
    NC, NS = info.num_cores, info.num_subcores
    NW = NC * NS
    b_per_s = B // NS
    full_tiles = D - NW
    mesh = plsc.VectorSubcoreMesh(core_axis_name="c", subcore_axis_name="s")

    @functools.partial(
        pl.kernel,
        mesh=mesh,
        compiler_params=pltpu.CompilerParams(
            use_tc_tiling_on_sc=True, needs_layout_passes=False),
        out_type=jax.ShapeDtypeStruct((D, B), jnp.float32),
        scratch_types=[
            pltpu.VMEM((2, b_per_s), jnp.int32),
            pltpu.VMEM((b_per_s,), jnp.int32),
            pltpu.VMEM((B,), jnp.int32),
            pltpu.VMEM((V,), jnp.float32),
            pltpu.VMEM((V,), jnp.float32),
            pltpu.VMEM((B,), jnp.float32),
            pltpu.VMEM((B,), jnp.float32),
            pltpu.VMEM_SHARED((B,), jnp.int32),
            pltpu.SemaphoreType.DMA,
            pltpu.SemaphoreType.DMA,
            pltpu.SemaphoreType.DMA,
        ],
    )
    def k(labt_hbm, tabt_hbm, out_hbm,
          lv, il_v, idx_v, rowa_v, rowb_v, outa_v, outb_v,
          idx_sh, sem_a, sem_b, sem_l):
        cid = lax.axis_index("c")
        sid = lax.axis_index("s")
        wid = sid * NC + cid
        sbase = sid * b_per_s
        cp_l = pltpu.async_copy(
            labt_hbm.at[:, pl.ds(sbase, b_per_s)], lv, sem_l)
        cp_a = pltpu.async_copy(tabt_hbm.at[wid], rowa_v, sem_a)

        @pl.when(wid < full_tiles)
        def _stage_b():
            pltpu.async_copy(tabt_hbm.at[wid + NW], rowb_v, sem_b)

        cp_l.wait()

        @plsc.parallel_loop(0, b_per_s // _L, unroll=2)
        def _idx(i):
            sl = pl.ds(i * _L, _L)
            il_v[sl] = lv[0, sl] * _NUM_OBJS + lv[1, sl]

        pltpu.sync_copy(il_v, idx_sh.at[pl.ds(sbase, b_per_s)])
        plsc.subcore_barrier()
        pltpu.sync_copy(idx_sh, idx_v)

        cp_a.wait()

        @pl.when(wid < full_tiles)
        def _wait_b():
            pltpu.make_async_copy(
                tabt_hbm.at[wid + NW], rowb_v, sem_b).wait()

        nchunk = 4
        csz = B // nchunk
        wb_a = []
        for h in range(nchunk):
            @plsc.parallel_loop(h * (csz // _L), (h + 1) * (csz // _L),
                                unroll=2)
            def _g2(i):
                sl = pl.ds(i * _L, _L)
                iv = idx_v[sl]
                outa_v[sl] = plsc.load_gather(rowa_v, [iv])
                outb_v[sl] = plsc.load_gather(rowb_v, [iv])

            hs = pl.ds(h * csz, csz)
            wb_a.append(pltpu.async_copy(
                outa_v.at[hs], out_hbm.at[wid, hs], sem_a))

            @pl.when(wid < full_tiles)
            def _wb_b():
                pltpu.async_copy(
                    outb_v.at[hs], out_hbm.at[wid + NW, hs], sem_b)

        for c in wb_a:
            c.wait()

        @pl.when(wid < full_tiles)
        def _wb_b_wait():
            for h in range(nchunk):
                hs = pl.ds(h * csz, csz)
                pltpu.make_async_copy(
                    outb_v.at[hs], out_hbm.at[wid + NW, hs], sem_b).wait()

    return k


def kernel(labels, table):
    B = labels.shape[0]
    V, D = table.shape
    labt = labels.T.astype(jnp.int32)
    tabt = table.T
    out_t = _make_gather(B, V, D)(labt, tabt)
    return out_t.T


# speedup vs baseline: 1.0091x; 1.0091x over previous
import functools

import jax
import jax.numpy as jnp
from jax import lax
from jax.experimental import pallas as pl
from jax.experimental.pallas import tpu as pltpu
from jax.experimental.pallas import tpu_sc as plsc

_NUM_OBJS = 151
_L = 16


@functools.lru_cache(maxsize=None)
def _make_gather(B, V, D):
    info = plsc.get_sparse_core_info()
    NC, NS = info.num_cores, info.num_subcores
    NW = NC * NS
    b_per_s = B // NS
    full_tiles = D - NW
    mesh = plsc.VectorSubcoreMesh(core_axis_name="c", subcore_axis_name="s")

    @functools.partial(
        pl.kernel,
        mesh=mesh,
        compiler_params=pltpu.CompilerParams(
            use_tc_tiling_on_sc=True, needs_layout_passes=False),
        out_type=jax.ShapeDtypeStruct((D, B), jnp.float32),
        scratch_types=[
            pltpu.VMEM((2, b_per_s), jnp.int32),
            pltpu.VMEM((b_per_s,), jnp.int32),
            pltpu.VMEM((B,), jnp.int32),
            pltpu.VMEM((V,), jnp.float32),
            pltpu.VMEM((V,), jnp.float32),
            pltpu.VMEM((B,), jnp.float32),
            pltpu.VMEM((B,), jnp.float32),
            pltpu.VMEM_SHARED((B,), jnp.int32),
            pltpu.SemaphoreType.DMA,
            pltpu.SemaphoreType.DMA,
            pltpu.SemaphoreType.DMA,
        ],
    )
    def k(labt_hbm, tabt_hbm, out_hbm,
          lv, il_v, idx_v, rowa_v, rowb_v, outa_v, outb_v,
          idx_sh, sem_a, sem_b, sem_l):
        cid = lax.axis_index("c")
        sid = lax.axis_index("s")
        wid = sid * NC + cid
        sbase = sid * b_per_s
        cp_l = pltpu.async_copy(
            labt_hbm.at[:, pl.ds(sbase, b_per_s)], lv, sem_l)
        cp_a = pltpu.async_copy(tabt_hbm.at[wid], rowa_v, sem_a)

        @pl.when(wid < full_tiles)
        def _stage_b():
            pltpu.async_copy(tabt_hbm.at[wid + NW], rowb_v, sem_b)

        cp_l.wait()

        @plsc.parallel_loop(0, b_per_s // _L, unroll=4)
        def _idx(i):
            sl = pl.ds(i * _L, _L)
            il_v[sl] = lv[0, sl] * _NUM_OBJS + lv[1, sl]

        pltpu.sync_copy(il_v, idx_sh.at[pl.ds(sbase, b_per_s)])
        plsc.subcore_barrier()
        pltpu.sync_copy(idx_sh, idx_v)

        cp_a.wait()

        @pl.when(wid < full_tiles)
        def _wait_b():
            pltpu.make_async_copy(
                tabt_hbm.at[wid + NW], rowb_v, sem_b).wait()

        nchunk = 4
        csz = B // nchunk
        wb_a = []
        for h in range(nchunk):
            @plsc.parallel_loop(h * (csz // _L), (h + 1) * (csz // _L),
                                unroll=4)
            def _g2(i):
                sl = pl.ds(i * _L, _L)
                iv = idx_v[sl]
                outa_v[sl] = plsc.load_gather(rowa_v, [iv])
                outb_v[sl] = plsc.load_gather(rowb_v, [iv])

            hs = pl.ds(h * csz, csz)
            wb_a.append(pltpu.async_copy(
                outa_v.at[hs], out_hbm.at[wid, hs], sem_a))

            @pl.when(wid < full_tiles)
            def _wb_b():
                pltpu.async_copy(
                    outb_v.at[hs], out_hbm.at[wid + NW, hs], sem_b)

        for c in wb_a:
            c.wait()

        @pl.when(wid < full_tiles)
        def _wb_b_wait():
            for h in range(nchunk):
                hs = pl.ds(h * csz, csz)
                pltpu.make_async_copy(
                    outb_v.at[hs], out_hbm.at[wid + NW, hs], sem_b).wait()

    return k


def kernel(labels, table):
    B = labels.shape[0]
    V, D = table.shape
    labt = labels.T.astype(jnp.int32)
    tabt = table.T
    out_t = _make_gather(B, V, D)(labt, tabt)
    return out_t.T
